# baseline (speedup 1.0000x reference)
import functools

import jax
import jax.numpy as jnp
from jax import lax
from jax.experimental import pallas as pl
from jax.experimental.pallas import tpu as pltpu
from jax.experimental.pallas import tpu_sc as plsc

N = 10000
E = 320000
D = 128
DE = 144
PW = 80
NEG = 0.2
NC, NS = 2, 16
NW = NC * NS
EPT = E // NW
K = 64
SUP = 4
NSUP = 40
CH = SUP * NSUP
NPT = N // NS
NP = 10240
BLK1 = 1024
BLK = 1000


MAXPOS = NP - 128


def _tc1_body(x_ref, w_ref, wb_ref, aw_ref, ab_ref,
              hext_ref, hbf_ref, st_ref, sn_ref, smax_ref):
    i = pl.program_id(0)
    hb = lax.dot_general(
        x_ref[...], w_ref[...], (((1,), (1,)), ((), ())),
        preferred_element_type=jnp.float32,
        precision=lax.Precision.HIGHEST) + wb_ref[...]
    extra = (lax.broadcasted_iota(jnp.int32, (BLK1, DE - D), 1) == 0)
    hx = jnp.concatenate([hb, extra.astype(jnp.float32)], axis=1)
    hext_ref[...] = hx

    def pack(g):
        a = hx[:, 32 * g:32 * g + 16]
        if 32 * g + 32 <= DE:
            b = hx[:, 32 * g + 16:32 * g + 32]
        else:
            b = jnp.zeros((BLK1, 16), jnp.float32)
        au = lax.bitcast_convert_type(
            a.astype(jnp.bfloat16), jnp.uint16).astype(jnp.uint32)
        bu = lax.bitcast_convert_type(
            b.astype(jnp.bfloat16), jnp.uint16).astype(jnp.uint32)
        return lax.bitcast_convert_type(au | (bu << 16), jnp.int32)

    hbf_ref[...] = jnp.concatenate([pack(g) for g in range(PW // 16)], axis=1)
    sv = lax.dot_general(
        aw_ref[...], hb, (((1,), (1,)), ((), ())),
        preferred_element_type=jnp.float32,
        precision=lax.Precision.HIGHEST) + ab_ref[...]
    st_ref[...] = sv[0]
    sn_ref[...] = sv[1]
    row = i * BLK1 + lax.broadcasted_iota(jnp.int32, (2, BLK1), 1)
    svm = jnp.where(row < N, sv, -jnp.inf)
    bmax = jnp.max(svm, axis=1)

    @pl.when(i == 0)
    def _():
        smax_ref[0] = -jnp.inf
        smax_ref[1] = -jnp.inf

    smax_ref[0] = jnp.maximum(smax_ref[0], bmax[0])
    smax_ref[1] = jnp.maximum(smax_ref[1], bmax[1])

    @pl.when(i == NP // BLK1 - 1)
    def _():
        off = pl.multiple_of(MAXPOS - i * BLK1, 128)
        st_ref[pl.ds(off, 128)] = jnp.full((128,), smax_ref[0], jnp.float32)
        sn_ref[pl.ds(off, 128)] = jnp.full((128,), smax_ref[1], jnp.float32)


def _tc2_body(agg_ref, hext_ref, g_ref, b_ref, o_ref):
    a = agg_ref[0] + agg_ref[1]
    den = a[:, D:D + 1] + 1e-16
    xx = a[:, :D] / den + hext_ref[:, :D]
    xx = jnp.where(xx > 0, xx, jnp.exp(jnp.minimum(xx, 0.0)) - 1.0)
    mean = jnp.mean(xx, axis=1, keepdims=True)
    xc = xx - mean
    var = jnp.mean(xc * xc, axis=1, keepdims=True)
    y = xc * lax.rsqrt(var + 1e-5)
    o_ref[...] = y * g_ref[...] + b_ref[...]


def _sc_body(tgt_hbm, nbr_hbm, st_hbm, sn_hbm, hext_hbm, out_hbm,
             s_t, s_n, c16_v,
             tA, nA, pA, tB, nB, pB, hbuf0, hbuf1, sbuf, agg,
             semiA, semiB, semh0, semh1, sems0, sems1):
    cid = lax.axis_index("c")
    sid = lax.axis_index("s")

    pltpu.sync_copy(st_hbm.at[pl.ds(0, N)], s_t)
    pltpu.sync_copy(sn_hbm.at[pl.ds(0, N)], s_n)

    pltpu.sync_copy(st_hbm.at[pl.ds(MAXPOS, 16)], c16_v)
    cbt = c16_v[...]
    pltpu.sync_copy(sn_hbm.at[pl.ds(MAXPOS, 16)], c16_v)
    cb = cbt + c16_v[...]
    C = jnp.where(cb > 0, cb, NEG * cb)

    def issue_idx(ss, t_b, n_b, semi):
        pltpu.async_copy(tgt_hbm.at[cid, sid, pl.ds(ss * SUP, SUP)], t_b, semi)
        pltpu.async_copy(nbr_hbm.at[cid, sid, pl.ds(ss * SUP, SUP)], n_b, semi)

    def wait_idx(ss, t_b, n_b, semi):
        pltpu.make_async_copy(
            tgt_hbm.at[cid, sid, pl.ds(ss * SUP, SUP)], t_b, semi).wait()
        pltpu.make_async_copy(
            nbr_hbm.at[cid, sid, pl.ds(ss * SUP, SUP)], n_b, semi).wait()

    def compute_p(ss, t_b, n_b, p_b):
        for cc in range(SUP):
            for g in range(K // 16):
                sl = pl.ds(g * 16, 16)
                e = (plsc.load_gather(s_t, [t_b[cc, sl]])
                     + plsc.load_gather(s_n, [n_b[cc, sl]]))
                e = jnp.where(e > 0, e, NEG * e)
                p = jnp.exp(e - C)
                eidx = (ss * (SUP * K) + (cc * K + g * 16)
                        + lax.iota(jnp.int32, 16))
                p_b[cc, sl] = jnp.where(eidx < EPT, p, 0.0)

    def issue_hext(n_b, cc, hbuf, semh):
        pltpu.async_copy(hext_hbm.at[n_b.at[cc]], hbuf, semh)

    def wait_hext(n_b, cc, hbuf, semh):
        pltpu.make_async_copy(hext_hbm.at[n_b.at[cc]], hbuf, semh).wait()

    def scale(p_b, cc, hbuf):
        cc16 = jnp.full((16,), cc, jnp.int32)

        def body(k, carry):
            pk = plsc.load_gather(p_b, [cc16, jnp.full((16,), k, jnp.int32)])
            for g in range(PW // 16):
                v = hbuf[k, pl.ds(16 * g, 16)]
                lo = lax.bitcast_convert_type(lax.shift_left(v, 16),
                                              jnp.float32)
                sbuf[k, pl.ds(32 * g, 16)] = lo * pk
                if 32 * g + 32 <= DE:
                    hi = lax.bitcast_convert_type(v & jnp.int32(-65536),
                                                  jnp.float32)
                    sbuf[k, pl.ds(32 * g + 16, 16)] = hi * pk
            return carry

        lax.fori_loop(0, K, body, 0)

    def issue_scatter(t_b, cc, sems):
        pltpu.async_copy(sbuf, agg.at[t_b.at[cc]], sems, add=True)

    def wait_scatter(t_b, cc, sems):
        pltpu.make_async_copy(sbuf, agg.at[t_b.at[cc]], sems).wait()

    zero16 = jnp.zeros((16,), jnp.float32)

    def zrow(r, carry):
        for m in range(DE // 16):
            sbuf[r, pl.ds(m * 16, 16)] = zero16
        return carry

    lax.fori_loop(0, K, zrow, 0)
    base = sid * NPT
    for i in range(NPT // K):
        pltpu.sync_copy(sbuf.at[pl.ds(0, K)],
                        agg.at[pl.ds(base + i * K, K)])
    rem = NPT - (NPT // K) * K
    if rem:
        pltpu.sync_copy(sbuf.at[pl.ds(0, rem)],
                        agg.at[pl.ds(base + NPT - rem, rem)])

    issue_idx(0, tA, nA, semiA)
    issue_idx(1, tB, nB, semiB)
    wait_idx(0, tA, nA, semiA)
    compute_p(0, tA, nA, pA)
    issue_hext(nA, 0, hbuf0, semh0)
    issue_hext(nA, 1, hbuf1, semh1)

    plsc.subcore_barrier()

    hbufs = ((hbuf0, semh0, sems0), (hbuf1, semh1, sems1))

    def run_super(sc, cur, nxt, semi_cur, semi_nxt):
        t_c, n_c, p_c = cur
        t_n, n_n, p_n = nxt
        for cc in (0, 1):
            hbuf, semh, sems = hbufs[cc]
            wait_hext(n_c, cc, hbuf, semh)
            scale(p_c, cc, hbuf)
            issue_scatter(t_c, cc, sems)
            wait_scatter(t_c, cc, sems)
            issue_hext(n_c, cc + 2, hbuf, semh)
        nxt_exists = sc + 1 < NSUP

        @pl.when(nxt_exists)
        def _():
            wait_idx(sc + 1, t_n, n_n, semi_nxt)

        for cc in (2, 3):
            hbuf, semh, sems = hbufs[cc % 2]
            wait_hext(n_c, cc, hbuf, semh)
            scale(p_c, cc, hbuf)
            issue_scatter(t_c, cc, sems)
            wait_scatter(t_c, cc, sems)

            @pl.when(nxt_exists)
            def _():
                issue_hext(n_n, cc - 2, hbuf, semh)

        @pl.when(nxt_exists)
        def _():
            compute_p(sc + 1, t_n, n_n, p_n)

        @pl.when(sc + 2 < NSUP)
        def _():
            issue_idx(sc + 2, t_c, n_c, semi_cur)

    bufsA = (tA, nA, pA)
    bufsB = (tB, nB, pB)

    def outer(j, carry):
        run_super(2 * j, bufsA, bufsB, semiA, semiB)
        run_super(2 * j + 1, bufsB, bufsA, semiB, semiA)
        return carry

    lax.fori_loop(0, NSUP // 2, outer, 0)

    plsc.subcore_barrier()
    pltpu.sync_copy(agg.at[pl.ds(base, NPT)],
                    out_hbm.at[cid, pl.ds(base, NPT)])


@functools.lru_cache(maxsize=1)
def _sc_edge_phase():
    mesh = plsc.VectorSubcoreMesh(core_axis_name="c", subcore_axis_name="s")
    return pl.kernel(
        _sc_body,
        out_type=jax.ShapeDtypeStruct((NC, N, DE), jnp.float32),
        mesh=mesh,
        compiler_params=pltpu.CompilerParams(use_tc_tiling_on_sc=False,
                                             needs_layout_passes=False),
        scratch_types=[
            pltpu.VMEM((N,), jnp.float32),
            pltpu.VMEM((N,), jnp.float32),
            pltpu.VMEM((16,), jnp.float32),
            pltpu.VMEM((SUP, K), jnp.int32),
            pltpu.VMEM((SUP, K), jnp.int32),
            pltpu.VMEM((SUP, K), jnp.float32),
            pltpu.VMEM((SUP, K), jnp.int32),
            pltpu.VMEM((SUP, K), jnp.int32),
            pltpu.VMEM((SUP, K), jnp.float32),
            pltpu.VMEM((K, PW), jnp.int32),
            pltpu.VMEM((K, PW), jnp.int32),
            pltpu.VMEM((K, DE), jnp.float32),
            pltpu.VMEM_SHARED((N, DE), jnp.float32),
            pltpu.SemaphoreType.DMA,
            pltpu.SemaphoreType.DMA,
            pltpu.SemaphoreType.DMA,
            pltpu.SemaphoreType.DMA,
            pltpu.SemaphoreType.DMA,
            pltpu.SemaphoreType.DMA,
        ],
    )


def kernel(node_features, edge_index, w_weight, w_bias, attn_weight,
           attn_bias, ln_gamma, ln_beta):
    x = node_features.astype(jnp.float32)
    tgt = edge_index[0].astype(jnp.int32).reshape(NW, EPT)
    nbr = edge_index[1].astype(jnp.int32).reshape(NW, EPT)
    pad = CH * K - EPT
    tgt4 = jnp.pad(tgt, ((0, 0), (0, pad))).reshape(NC, NS, CH, K)
    nbr4 = jnp.pad(nbr, ((0, 0), (0, pad))).reshape(NC, NS, CH, K)
    aw2 = attn_weight.reshape(2, D).astype(jnp.float32)
    ab2 = jnp.stack([attn_bias[0].astype(jnp.float32),
                     jnp.zeros((), jnp.float32)]).reshape(2, 1)
    wb = w_bias.reshape(1, D).astype(jnp.float32)

    xp = jnp.pad(x, ((0, NP - N), (0, 0)))
    hext, hbf, s_t, s_n = pl.pallas_call(
        _tc1_body,
        grid=(NP // BLK1,),
        in_specs=[
            pl.BlockSpec((BLK1, D), lambda i: (i, 0)),
            pl.BlockSpec((D, D), lambda i: (0, 0)),
            pl.BlockSpec((1, D), lambda i: (0, 0)),
            pl.BlockSpec((2, D), lambda i: (0, 0)),
            pl.BlockSpec((2, 1), lambda i: (0, 0)),
        ],
        out_specs=[
            pl.BlockSpec((BLK1, DE), lambda i: (i, 0)),
            pl.BlockSpec((BLK1, PW), lambda i: (i, 0)),
            pl.BlockSpec((BLK1,), lambda i: (i,)),
            pl.BlockSpec((BLK1,), lambda i: (i,)),
        ],
        out_shape=[
            jax.ShapeDtypeStruct((NP, DE), jnp.float32),
            jax.ShapeDtypeStruct((NP, PW), jnp.int32),
            jax.ShapeDtypeStruct((NP,), jnp.float32),
            jax.ShapeDtypeStruct((NP,), jnp.float32),
        ],
        scratch_shapes=[pltpu.SMEM((2,), jnp.float32)],
    )(xp, w_weight.astype(jnp.float32), wb, aw2, ab2)

    agg = _sc_edge_phase()(tgt4, nbr4, s_t, s_n, hbf)

    out = pl.pallas_call(
        _tc2_body,
        grid=(N // BLK,),
        in_specs=[
            pl.BlockSpec((2, BLK, DE), lambda i: (0, i, 0)),
            pl.BlockSpec((BLK, DE), lambda i: (i, 0)),
            pl.BlockSpec((1, D), lambda i: (0, 0)),
            pl.BlockSpec((1, D), lambda i: (0, 0)),
        ],
        out_specs=pl.BlockSpec((BLK, D), lambda i: (i, 0)),
        out_shape=jax.ShapeDtypeStruct((N, D), jnp.float32),
    )(agg, hext, ln_gamma.reshape(1, D).astype(jnp.float32),
      ln_beta.reshape(1, D).astype(jnp.float32))
    return out


# speedup vs baseline: 1.1125x; 1.1125x over previous
import functools

import jax
import jax.numpy as jnp
from jax import lax
from jax.experimental import pallas as pl
from jax.experimental.pallas import tpu as pltpu
from jax.experimental.pallas import tpu_sc as plsc

N = 10000
E = 320000
D = 128
DE = 144
NEG = 0.2
NC, NS = 2, 16
NW = NC * NS
EPT = E // NW
K = 64
SUP = 4
NSUP = 40
CH = SUP * NSUP
NPT = N // NS
NP = 10240
BLK1 = 1024
BLK = 1000


MAXPOS = NP - 128


def _tc1_body(x_ref, w_ref, wb_ref, aw_ref, ab_ref,
              hext_ref, st_ref, sn_ref, smax_ref):
    i = pl.program_id(0)
    hb = lax.dot_general(
        x_ref[...], w_ref[...], (((1,), (1,)), ((), ())),
        preferred_element_type=jnp.float32,
        precision=lax.Precision.HIGHEST) + wb_ref[...]
    extra = (lax.broadcasted_iota(jnp.int32, (BLK1, DE - D), 1) == 0)
    hext_ref[...] = jnp.concatenate([hb, extra.astype(jnp.float32)], axis=1)
    sv = lax.dot_general(
        aw_ref[...], hb, (((1,), (1,)), ((), ())),
        preferred_element_type=jnp.float32,
        precision=lax.Precision.HIGHEST) + ab_ref[...]
    st_ref[...] = sv[0]
    sn_ref[...] = sv[1]
    row = i * BLK1 + lax.broadcasted_iota(jnp.int32, (2, BLK1), 1)
    svm = jnp.where(row < N, sv, -jnp.inf)
    bmax = jnp.max(svm, axis=1)

    @pl.when(i == 0)
    def _():
        smax_ref[0] = -jnp.inf
        smax_ref[1] = -jnp.inf

    smax_ref[0] = jnp.maximum(smax_ref[0], bmax[0])
    smax_ref[1] = jnp.maximum(smax_ref[1], bmax[1])

    @pl.when(i == NP // BLK1 - 1)
    def _():
        off = pl.multiple_of(MAXPOS - i * BLK1, 128)
        st_ref[pl.ds(off, 128)] = jnp.full((128,), smax_ref[0], jnp.float32)
        sn_ref[pl.ds(off, 128)] = jnp.full((128,), smax_ref[1], jnp.float32)


def _tc2_body(agg_ref, hext_ref, g_ref, b_ref, o_ref):
    a = agg_ref[0] + agg_ref[1]
    den = a[:, D:D + 1] + 1e-16
    xx = a[:, :D] / den + hext_ref[:, :D]
    xx = jnp.where(xx > 0, xx, jnp.exp(jnp.minimum(xx, 0.0)) - 1.0)
    mean = jnp.mean(xx, axis=1, keepdims=True)
    xc = xx - mean
    var = jnp.mean(xc * xc, axis=1, keepdims=True)
    y = xc * lax.rsqrt(var + 1e-5)
    o_ref[...] = y * g_ref[...] + b_ref[...]


def _sc_body(tgt_hbm, nbr_hbm, st_hbm, sn_hbm, hext_hbm, out_hbm,
             s_t, s_n, c16_v,
             tA, nA, pA, tB, nB, pB, hbuf0, hbuf1, agg,
             semiA, semiB, semh0a, semh0b, semh1a, semh1b):
    cid = lax.axis_index("c")
    sid = lax.axis_index("s")

    pltpu.sync_copy(st_hbm.at[pl.ds(0, N)], s_t)
    pltpu.sync_copy(sn_hbm.at[pl.ds(0, N)], s_n)

    pltpu.sync_copy(st_hbm.at[pl.ds(MAXPOS, 16)], c16_v)
    cbt = c16_v[...]
    pltpu.sync_copy(sn_hbm.at[pl.ds(MAXPOS, 16)], c16_v)
    cb = cbt + c16_v[...]
    C = jnp.where(cb > 0, cb, NEG * cb)

    def issue_idx(ss, t_b, n_b, semi):
        pltpu.async_copy(tgt_hbm.at[cid, sid, pl.ds(ss * SUP, SUP)], t_b, semi)
        pltpu.async_copy(nbr_hbm.at[cid, sid, pl.ds(ss * SUP, SUP)], n_b, semi)

    def wait_idx(ss, t_b, n_b, semi):
        pltpu.make_async_copy(
            tgt_hbm.at[cid, sid, pl.ds(ss * SUP, SUP)], t_b, semi).wait()
        pltpu.make_async_copy(
            nbr_hbm.at[cid, sid, pl.ds(ss * SUP, SUP)], n_b, semi).wait()

    def compute_p(ss, t_b, n_b, p_b):
        for cc in range(SUP):
            for g in range(K // 16):
                sl = pl.ds(g * 16, 16)
                e = (plsc.load_gather(s_t, [t_b[cc, sl]])
                     + plsc.load_gather(s_n, [n_b[cc, sl]]))
                e = jnp.where(e > 0, e, NEG * e)
                p = jnp.exp(e - C)
                eidx = (ss * (SUP * K) + (cc * K + g * 16)
                        + lax.iota(jnp.int32, 16))
                p_b[cc, sl] = jnp.where(eidx < EPT, p, 0.0)

    HK = K // 2

    def issue_hext(n_b, cc, hbuf, semh):
        pltpu.async_copy(hext_hbm.at[n_b.at[cc, pl.ds(0, HK)]],
                         hbuf.at[pl.ds(0, HK)], semh[0])
        pltpu.async_copy(hext_hbm.at[n_b.at[cc, pl.ds(HK, HK)]],
                         hbuf.at[pl.ds(HK, HK)], semh[1])

    def wait_hext(n_b, cc, hbuf, semh):
        pltpu.make_async_copy(hext_hbm.at[n_b.at[cc, pl.ds(0, HK)]],
                              hbuf.at[pl.ds(0, HK)], semh[0]).wait()
        pltpu.make_async_copy(hext_hbm.at[n_b.at[cc, pl.ds(HK, HK)]],
                              hbuf.at[pl.ds(HK, HK)], semh[1]).wait()

    def scale(p_b, cc, hbuf):
        cc16 = jnp.full((16,), cc, jnp.int32)

        def body(k, carry):
            pk = plsc.load_gather(p_b, [cc16, jnp.full((16,), k, jnp.int32)])
            for m in range(DE // 16):
                sl = pl.ds(m * 16, 16)
                hbuf[k, sl] = hbuf[k, sl] * pk
            return carry

        lax.fori_loop(0, K, body, 0)

    def scatter(t_b, cc, hbuf):
        pltpu.sync_copy(hbuf, agg.at[t_b.at[cc]], add=True)

    zero16 = jnp.zeros((16,), jnp.float32)

    def zrow(r, carry):
        for m in range(DE // 16):
            hbuf0[r, pl.ds(m * 16, 16)] = zero16
        return carry

    lax.fori_loop(0, K, zrow, 0)
    base = sid * NPT
    for i in range(NPT // K):
        pltpu.sync_copy(hbuf0.at[pl.ds(0, K)],
                        agg.at[pl.ds(base + i * K, K)])
    rem = NPT - (NPT // K) * K
    if rem:
        pltpu.sync_copy(hbuf0.at[pl.ds(0, rem)],
                        agg.at[pl.ds(base + NPT - rem, rem)])

    issue_idx(0, tA, nA, semiA)
    issue_idx(1, tB, nB, semiB)
    wait_idx(0, tA, nA, semiA)
    compute_p(0, tA, nA, pA)
    semh0 = (semh0a, semh0b)
    semh1 = (semh1a, semh1b)
    issue_hext(nA, 0, hbuf0, semh0)
    issue_hext(nA, 1, hbuf1, semh1)

    plsc.subcore_barrier()

    hbufs = ((hbuf0, semh0), (hbuf1, semh1))

    def run_super(sc, cur, nxt, semi_cur, semi_nxt):
        t_c, n_c, p_c = cur
        t_n, n_n, p_n = nxt
        for cc in (0, 1):
            hbuf, semh = hbufs[cc]
            wait_hext(n_c, cc, hbuf, semh)
            scale(p_c, cc, hbuf)
            scatter(t_c, cc, hbuf)
            issue_hext(n_c, cc + 2, hbuf, semh)
        nxt_exists = sc + 1 < NSUP

        @pl.when(nxt_exists)
        def _():
            wait_idx(sc + 1, t_n, n_n, semi_nxt)

        for cc in (2, 3):
            hbuf, semh = hbufs[cc % 2]
            wait_hext(n_c, cc, hbuf, semh)
            scale(p_c, cc, hbuf)
            scatter(t_c, cc, hbuf)

            @pl.when(nxt_exists)
            def _():
                issue_hext(n_n, cc - 2, hbuf, semh)

        @pl.when(nxt_exists)
        def _():
            compute_p(sc + 1, t_n, n_n, p_n)

        @pl.when(sc + 2 < NSUP)
        def _():
            issue_idx(sc + 2, t_c, n_c, semi_cur)

    bufsA = (tA, nA, pA)
    bufsB = (tB, nB, pB)

    def outer(j, carry):
        run_super(2 * j, bufsA, bufsB, semiA, semiB)
        run_super(2 * j + 1, bufsB, bufsA, semiB, semiA)
        return carry

    lax.fori_loop(0, NSUP // 2, outer, 0)

    plsc.subcore_barrier()
    pltpu.sync_copy(agg.at[pl.ds(base, NPT)],
                    out_hbm.at[cid, pl.ds(base, NPT)])


@functools.lru_cache(maxsize=1)
def _sc_edge_phase():
    mesh = plsc.VectorSubcoreMesh(core_axis_name="c", subcore_axis_name="s")
    return pl.kernel(
        _sc_body,
        out_type=jax.ShapeDtypeStruct((NC, N, DE), jnp.float32),
        mesh=mesh,
        compiler_params=pltpu.CompilerParams(use_tc_tiling_on_sc=False,
                                             needs_layout_passes=False),
        scratch_types=[
            pltpu.VMEM((N,), jnp.float32),
            pltpu.VMEM((N,), jnp.float32),
            pltpu.VMEM((16,), jnp.float32),
            pltpu.VMEM((SUP, K), jnp.int32),
            pltpu.VMEM((SUP, K), jnp.int32),
            pltpu.VMEM((SUP, K), jnp.float32),
            pltpu.VMEM((SUP, K), jnp.int32),
            pltpu.VMEM((SUP, K), jnp.int32),
            pltpu.VMEM((SUP, K), jnp.float32),
            pltpu.VMEM((K, DE), jnp.float32),
            pltpu.VMEM((K, DE), jnp.float32),
            pltpu.VMEM_SHARED((N, DE), jnp.float32),
            pltpu.SemaphoreType.DMA,
            pltpu.SemaphoreType.DMA,
            pltpu.SemaphoreType.DMA,
            pltpu.SemaphoreType.DMA,
            pltpu.SemaphoreType.DMA,
            pltpu.SemaphoreType.DMA,
        ],
    )


def kernel(node_features, edge_index, w_weight, w_bias, attn_weight,
           attn_bias, ln_gamma, ln_beta):
    x = node_features.astype(jnp.float32)
    tgt = edge_index[0].astype(jnp.int32).reshape(NW, EPT)
    nbr = edge_index[1].astype(jnp.int32).reshape(NW, EPT)
    pad = CH * K - EPT
    tgt4 = jnp.pad(tgt, ((0, 0), (0, pad))).reshape(NC, NS, CH, K)
    nbr4 = jnp.pad(nbr, ((0, 0), (0, pad))).reshape(NC, NS, CH, K)
    aw2 = attn_weight.reshape(2, D).astype(jnp.float32)
    ab2 = jnp.stack([attn_bias[0].astype(jnp.float32),
                     jnp.zeros((), jnp.float32)]).reshape(2, 1)
    wb = w_bias.reshape(1, D).astype(jnp.float32)

    xp = jnp.pad(x, ((0, NP - N), (0, 0)))
    hext, s_t, s_n = pl.pallas_call(
        _tc1_body,
        grid=(NP // BLK1,),
        in_specs=[
            pl.BlockSpec((BLK1, D), lambda i: (i, 0)),
            pl.BlockSpec((D, D), lambda i: (0, 0)),
            pl.BlockSpec((1, D), lambda i: (0, 0)),
            pl.BlockSpec((2, D), lambda i: (0, 0)),
            pl.BlockSpec((2, 1), lambda i: (0, 0)),
        ],
        out_specs=[
            pl.BlockSpec((BLK1, DE), lambda i: (i, 0)),
            pl.BlockSpec((BLK1,), lambda i: (i,)),
            pl.BlockSpec((BLK1,), lambda i: (i,)),
        ],
        out_shape=[
            jax.ShapeDtypeStruct((NP, DE), jnp.float32),
            jax.ShapeDtypeStruct((NP,), jnp.float32),
            jax.ShapeDtypeStruct((NP,), jnp.float32),
        ],
        scratch_shapes=[pltpu.SMEM((2,), jnp.float32)],
    )(xp, w_weight.astype(jnp.float32), wb, aw2, ab2)

    agg = _sc_edge_phase()(tgt4, nbr4, s_t, s_n, hext)

    out = pl.pallas_call(
        _tc2_body,
        grid=(N // BLK,),
        in_specs=[
            pl.BlockSpec((2, BLK, DE), lambda i: (0, i, 0)),
            pl.BlockSpec((BLK, DE), lambda i: (i, 0)),
            pl.BlockSpec((1, D), lambda i: (0, 0)),
            pl.BlockSpec((1, D), lambda i: (0, 0)),
        ],
        out_specs=pl.BlockSpec((BLK, D), lambda i: (i, 0)),
        out_shape=jax.ShapeDtypeStruct((N, D), jnp.float32),
    )(agg, hext, ln_gamma.reshape(1, D).astype(jnp.float32),
      ln_beta.reshape(1, D).astype(jnp.float32))
    return out


# speedup vs baseline: 1.1129x; 1.0004x over previous
import functools

import jax
import jax.numpy as jnp
from jax import lax
from jax.experimental import pallas as pl
from jax.experimental.pallas import tpu as pltpu
from jax.experimental.pallas import tpu_sc as plsc

N = 10000
E = 320000
D = 128
DE = 144
NEG = 0.2
NC, NS = 2, 16
NW = NC * NS
EPT = E // NW
K = 64
SUP = 4
NSUP = 40
CH = SUP * NSUP
NPT = N // NS
NP = 10240
BLK1 = 1024
BLK = 1000


MAXPOS = NP - 128


def _tc1_body(x_ref, w_ref, wb_ref, aw_ref, ab_ref,
              hext_ref, st_ref, sn_ref, smax_ref):
    i = pl.program_id(0)
    hb = lax.dot_general(
        x_ref[...], w_ref[...], (((1,), (1,)), ((), ())),
        preferred_element_type=jnp.float32,
        precision=lax.Precision.HIGHEST) + wb_ref[...]
    extra = (lax.broadcasted_iota(jnp.int32, (BLK1, DE - D), 1) == 0)
    hext_ref[...] = jnp.concatenate([hb, extra.astype(jnp.float32)], axis=1)
    sv = lax.dot_general(
        aw_ref[...], hb, (((1,), (1,)), ((), ())),
        preferred_element_type=jnp.float32,
        precision=lax.Precision.HIGHEST) + ab_ref[...]
    st_ref[...] = sv[0]
    sn_ref[...] = sv[1]
    row = i * BLK1 + lax.broadcasted_iota(jnp.int32, (2, BLK1), 1)
    svm = jnp.where(row < N, sv, -jnp.inf)
    bmax = jnp.max(svm, axis=1)

    @pl.when(i == 0)
    def _():
        smax_ref[0] = -jnp.inf
        smax_ref[1] = -jnp.inf

    smax_ref[0] = jnp.maximum(smax_ref[0], bmax[0])
    smax_ref[1] = jnp.maximum(smax_ref[1], bmax[1])

    @pl.when(i == NP // BLK1 - 1)
    def _():
        off = pl.multiple_of(MAXPOS - i * BLK1, 128)
        st_ref[pl.ds(off, 128)] = jnp.full((128,), smax_ref[0], jnp.float32)
        sn_ref[pl.ds(off, 128)] = jnp.full((128,), smax_ref[1], jnp.float32)


def _tc2_body(agg_ref, hext_ref, g_ref, b_ref, o_ref):
    a = agg_ref[0] + agg_ref[1]
    den = a[:, D:D + 1] + 1e-16
    xx = a[:, :D] / den + hext_ref[:, :D]
    xx = jnp.where(xx > 0, xx, jnp.exp(jnp.minimum(xx, 0.0)) - 1.0)
    mean = jnp.mean(xx, axis=1, keepdims=True)
    xc = xx - mean
    var = jnp.mean(xc * xc, axis=1, keepdims=True)
    y = xc * lax.rsqrt(var + 1e-5)
    o_ref[...] = y * g_ref[...] + b_ref[...]


def _sc_body(tgt_hbm, nbr_hbm, st_hbm, sn_hbm, hext_hbm, out_hbm,
             s_t, s_n, c16_v,
             tA, nA, pA, tB, nB, pB, hbuf0, hbuf1, agg,
             semiA, semiB, semh0, semh1):
    cid = lax.axis_index("c")
    sid = lax.axis_index("s")

    pltpu.sync_copy(st_hbm.at[pl.ds(0, N)], s_t)
    pltpu.sync_copy(sn_hbm.at[pl.ds(0, N)], s_n)

    pltpu.sync_copy(st_hbm.at[pl.ds(MAXPOS, 16)], c16_v)
    cbt = c16_v[...]
    pltpu.sync_copy(sn_hbm.at[pl.ds(MAXPOS, 16)], c16_v)
    cb = cbt + c16_v[...]
    C = jnp.where(cb > 0, cb, NEG * cb)

    def issue_idx(ss, t_b, n_b, semi):
        pltpu.async_copy(tgt_hbm.at[cid, sid, pl.ds(ss * SUP, SUP)], t_b, semi)
        pltpu.async_copy(nbr_hbm.at[cid, sid, pl.ds(ss * SUP, SUP)], n_b, semi)

    def wait_idx(ss, t_b, n_b, semi):
        pltpu.make_async_copy(
            tgt_hbm.at[cid, sid, pl.ds(ss * SUP, SUP)], t_b, semi).wait()
        pltpu.make_async_copy(
            nbr_hbm.at[cid, sid, pl.ds(ss * SUP, SUP)], n_b, semi).wait()

    def compute_p(ss, t_b, n_b, p_b):
        for cc in range(SUP):
            for g in range(K // 16):
                sl = pl.ds(g * 16, 16)
                e = (plsc.load_gather(s_t, [t_b[cc, sl]])
                     + plsc.load_gather(s_n, [n_b[cc, sl]]))
                e = jnp.where(e > 0, e, NEG * e)
                p = jnp.exp(e - C)
                eidx = (ss * (SUP * K) + (cc * K + g * 16)
                        + lax.iota(jnp.int32, 16))
                p_b[cc, sl] = jnp.where(eidx < EPT, p, 0.0)

    def issue_hext(n_b, cc, hbuf, semh):
        pltpu.async_copy(hext_hbm.at[n_b.at[cc]], hbuf, semh)

    def wait_hext(n_b, cc, hbuf, semh):
        pltpu.make_async_copy(hext_hbm.at[n_b.at[cc]], hbuf, semh).wait()

    def scale(p_b, cc, hbuf):
        cc16 = jnp.full((16,), cc, jnp.int32)

        def body(kk, carry):
            k = 4 * kk
            kv = jnp.full((16,), k, jnp.int32)
            for u in range(4):
                pk = plsc.load_gather(p_b, [cc16, kv + u])
                for m in range(DE // 16):
                    sl = pl.ds(m * 16, 16)
                    hbuf[k + u, sl] = hbuf[k + u, sl] * pk
            return carry

        lax.fori_loop(0, K // 4, body, 0)

    def scatter(t_b, cc, hbuf):
        pltpu.sync_copy(hbuf, agg.at[t_b.at[cc]], add=True)

    zero16 = jnp.zeros((16,), jnp.float32)

    def zrow(r, carry):
        for m in range(DE // 16):
            hbuf0[r, pl.ds(m * 16, 16)] = zero16
        return carry

    lax.fori_loop(0, K, zrow, 0)
    base = sid * NPT
    for i in range(NPT // K):
        pltpu.sync_copy(hbuf0.at[pl.ds(0, K)],
                        agg.at[pl.ds(base + i * K, K)])
    rem = NPT - (NPT // K) * K
    if rem:
        pltpu.sync_copy(hbuf0.at[pl.ds(0, rem)],
                        agg.at[pl.ds(base + NPT - rem, rem)])

    issue_idx(0, tA, nA, semiA)
    issue_idx(1, tB, nB, semiB)
    wait_idx(0, tA, nA, semiA)
    compute_p(0, tA, nA, pA)
    issue_hext(nA, 0, hbuf0, semh0)
    issue_hext(nA, 1, hbuf1, semh1)

    plsc.subcore_barrier()

    hbufs = ((hbuf0, semh0), (hbuf1, semh1))

    def run_super(sc, cur, nxt, semi_cur, semi_nxt):
        t_c, n_c, p_c = cur
        t_n, n_n, p_n = nxt
        for cc in (0, 1):
            hbuf, semh = hbufs[cc]
            wait_hext(n_c, cc, hbuf, semh)
            scale(p_c, cc, hbuf)
            scatter(t_c, cc, hbuf)
            issue_hext(n_c, cc + 2, hbuf, semh)
        nxt_exists = sc + 1 < NSUP

        @pl.when(nxt_exists)
        def _():
            wait_idx(sc + 1, t_n, n_n, semi_nxt)

        for cc in (2, 3):
            hbuf, semh = hbufs[cc % 2]
            wait_hext(n_c, cc, hbuf, semh)
            scale(p_c, cc, hbuf)
            scatter(t_c, cc, hbuf)

            @pl.when(nxt_exists)
            def _():
                issue_hext(n_n, cc - 2, hbuf, semh)

        @pl.when(nxt_exists)
        def _():
            compute_p(sc + 1, t_n, n_n, p_n)

        @pl.when(sc + 2 < NSUP)
        def _():
            issue_idx(sc + 2, t_c, n_c, semi_cur)

    bufsA = (tA, nA, pA)
    bufsB = (tB, nB, pB)

    def outer(j, carry):
        run_super(2 * j, bufsA, bufsB, semiA, semiB)
        run_super(2 * j + 1, bufsB, bufsA, semiB, semiA)
        return carry

    lax.fori_loop(0, NSUP // 2, outer, 0)

    plsc.subcore_barrier()
    pltpu.sync_copy(agg.at[pl.ds(base, NPT)],
                    out_hbm.at[cid, pl.ds(base, NPT)])


@functools.lru_cache(maxsize=1)
def _sc_edge_phase():
    mesh = plsc.VectorSubcoreMesh(core_axis_name="c", subcore_axis_name="s")
    return pl.kernel(
        _sc_body,
        out_type=jax.ShapeDtypeStruct((NC, N, DE), jnp.float32),
        mesh=mesh,
        compiler_params=pltpu.CompilerParams(use_tc_tiling_on_sc=False,
                                             needs_layout_passes=False),
        scratch_types=[
            pltpu.VMEM((N,), jnp.float32),
            pltpu.VMEM((N,), jnp.float32),
            pltpu.VMEM((16,), jnp.float32),
            pltpu.VMEM((SUP, K), jnp.int32),
            pltpu.VMEM((SUP, K), jnp.int32),
            pltpu.VMEM((SUP, K), jnp.float32),
            pltpu.VMEM((SUP, K), jnp.int32),
            pltpu.VMEM((SUP, K), jnp.int32),
            pltpu.VMEM((SUP, K), jnp.float32),
            pltpu.VMEM((K, DE), jnp.float32),
            pltpu.VMEM((K, DE), jnp.float32),
            pltpu.VMEM_SHARED((N, DE), jnp.float32),
            pltpu.SemaphoreType.DMA,
            pltpu.SemaphoreType.DMA,
            pltpu.SemaphoreType.DMA,
            pltpu.SemaphoreType.DMA,
        ],
    )


def kernel(node_features, edge_index, w_weight, w_bias, attn_weight,
           attn_bias, ln_gamma, ln_beta):
    x = node_features.astype(jnp.float32)
    tgt = edge_index[0].astype(jnp.int32).reshape(NW, EPT)
    nbr = edge_index[1].astype(jnp.int32).reshape(NW, EPT)
    pad = CH * K - EPT
    tgt4 = jnp.pad(tgt, ((0, 0), (0, pad))).reshape(NC, NS, CH, K)
    nbr4 = jnp.pad(nbr, ((0, 0), (0, pad))).reshape(NC, NS, CH, K)
    aw2 = attn_weight.reshape(2, D).astype(jnp.float32)
    ab2 = jnp.stack([attn_bias[0].astype(jnp.float32),
                     jnp.zeros((), jnp.float32)]).reshape(2, 1)
    wb = w_bias.reshape(1, D).astype(jnp.float32)

    xp = jnp.pad(x, ((0, NP - N), (0, 0)))
    hext, s_t, s_n = pl.pallas_call(
        _tc1_body,
        grid=(NP // BLK1,),
        in_specs=[
            pl.BlockSpec((BLK1, D), lambda i: (i, 0)),
            pl.BlockSpec((D, D), lambda i: (0, 0)),
            pl.BlockSpec((1, D), lambda i: (0, 0)),
            pl.BlockSpec((2, D), lambda i: (0, 0)),
            pl.BlockSpec((2, 1), lambda i: (0, 0)),
        ],
        out_specs=[
            pl.BlockSpec((BLK1, DE), lambda i: (i, 0)),
            pl.BlockSpec((BLK1,), lambda i: (i,)),
            pl.BlockSpec((BLK1,), lambda i: (i,)),
        ],
        out_shape=[
            jax.ShapeDtypeStruct((NP, DE), jnp.float32),
            jax.ShapeDtypeStruct((NP,), jnp.float32),
            jax.ShapeDtypeStruct((NP,), jnp.float32),
        ],
        scratch_shapes=[pltpu.SMEM((2,), jnp.float32)],
    )(xp, w_weight.astype(jnp.float32), wb, aw2, ab2)

    agg = _sc_edge_phase()(tgt4, nbr4, s_t, s_n, hext)

    out = pl.pallas_call(
        _tc2_body,
        grid=(N // BLK,),
        in_specs=[
            pl.BlockSpec((2, BLK, DE), lambda i: (0, i, 0)),
            pl.BlockSpec((BLK, DE), lambda i: (i, 0)),
            pl.BlockSpec((1, D), lambda i: (0, 0)),
            pl.BlockSpec((1, D), lambda i: (0, 0)),
        ],
        out_specs=pl.BlockSpec((BLK, D), lambda i: (i, 0)),
        out_shape=jax.ShapeDtypeStruct((N, D), jnp.float32),
    )(agg, hext, ln_gamma.reshape(1, D).astype(jnp.float32),
      ln_beta.reshape(1, D).astype(jnp.float32))
    return out
